# baseline (speedup 1.0000x reference)
import jax
import jax.numpy as jnp
from jax import lax
from jax.experimental import pallas as pl
from jax.experimental.pallas import tpu as pltpu
from jax.experimental.pallas import tpu_sc as plsc

_B, _S, _D = 4, 4096, 1024
_NW = 32
_SPW = _S // _NW
_K = 16
_NCH = _SPW // _K


def _emb_body(tok_ref, emb_ref, pos_ref, out_ref, idx_v, pos_v, emb_v,
              gsem0, gsem1, gsem2, gsem3, osem):
    gsems = (gsem0, gsem1, gsem2, gsem3)
    cid = lax.axis_index("core")
    sid = lax.axis_index("subcore")
    wid = sid * 2 + cid
    s_base = wid * _SPW

    pltpu.sync_copy(tok_ref.at[wid], idx_v)

    def add_pos(par):
        def row(r, carry):
            for j in range(_D // 16):
                sl = pl.ds(j * 16, 16)
                emb_v[par, r, sl] = emb_v[par, r, sl] + pos_v[r, sl]
            return carry
        lax.fori_loop(0, _K, row, 0)

    def gather(b, c):
        return pltpu.async_copy(emb_ref.at[idx_v.at[b, c]], emb_v.at[b],
                                gsems[b])

    def chunk(c, carry):
        s0 = s_base + c * _K
        gs = [gather(b, c) for b in range(_B)]
        pltpu.sync_copy(pos_ref.at[pl.ds(s0, _K)], pos_v)
        outs = []
        for b in range(_B):
            gs[b].wait()
            add_pos(b)
            outs.append(pltpu.async_copy(emb_v.at[b],
                                         out_ref.at[b, pl.ds(s0, _K)], osem))
        for o in outs:
            o.wait()
        return carry

    lax.fori_loop(0, _NCH, chunk, 0)


def kernel(tokens, emb_table, pos_table):
    tok = (tokens.astype(jnp.int32)
           .reshape(_B, _NW, _NCH, _K)
           .transpose(1, 0, 2, 3))
    mesh = plsc.VectorSubcoreMesh(core_axis_name="core",
                                  subcore_axis_name="subcore")
    f = pl.kernel(
        _emb_body,
        out_type=jax.ShapeDtypeStruct((_B, _S, _D), jnp.float32),
        mesh=mesh,
        scratch_types=[
            pltpu.VMEM((_B, _NCH, _K), jnp.int32),
            pltpu.VMEM((_K, _D), jnp.float32),
            pltpu.VMEM((_B, _K, _D), jnp.float32),
            pltpu.SemaphoreType.DMA,
            pltpu.SemaphoreType.DMA,
            pltpu.SemaphoreType.DMA,
            pltpu.SemaphoreType.DMA,
            pltpu.SemaphoreType.DMA,
        ],
    )
    return f(tok, emb_table, pos_table)


# speedup vs baseline: 1.5508x; 1.5508x over previous
import jax
import jax.numpy as jnp
from jax import lax
from jax.experimental import pallas as pl
from jax.experimental.pallas import tpu as pltpu
from jax.experimental.pallas import tpu_sc as plsc

_B, _S, _D = 4, 4096, 1024
_NW = 32
_SPW = _S // _NW
_K = 16
_NCH = _SPW // _K


def _emb_body(tok_ref, emb_ref, pos_ref, out_ref, idx_v, pos_v, emb_v,
              gsem0, gsem1, gsem2, gsem3, osem):
    gsems = (gsem0, gsem1, gsem2, gsem3)
    cid = lax.axis_index("core")
    sid = lax.axis_index("subcore")
    wid = sid * 2 + cid
    s_base = wid * _SPW

    pltpu.sync_copy(tok_ref.at[wid], idx_v)

    def add_pair(b0, b1):
        def row(r, carry):
            for h in range(2):
                base = h * (_D // 2)
                pv = [pos_v[r, pl.ds(base + j * 16, 16)]
                      for j in range(_D // 32)]
                for b in (b0, b1):
                    for j in range(_D // 32):
                        sl = pl.ds(base + j * 16, 16)
                        emb_v[b, r, sl] = emb_v[b, r, sl] + pv[j]
            return carry
        lax.fori_loop(0, _K, row, 0)

    def gather(b, c):
        return pltpu.async_copy(emb_ref.at[idx_v.at[b, c]], emb_v.at[b],
                                gsems[b])

    def chunk(c, carry):
        s0 = s_base + c * _K
        g0, g1 = gather(0, c), gather(1, c)
        pltpu.sync_copy(pos_ref.at[pl.ds(s0, _K)], pos_v)
        g0.wait()
        g1.wait()
        g2, g3 = gather(2, c), gather(3, c)
        add_pair(0, 1)
        outs = [pltpu.async_copy(emb_v.at[b],
                                 out_ref.at[b, pl.ds(s0, _K)], osem)
                for b in (0, 1)]
        g2.wait()
        g3.wait()
        add_pair(2, 3)
        outs += [pltpu.async_copy(emb_v.at[b],
                                  out_ref.at[b, pl.ds(s0, _K)], osem)
                 for b in (2, 3)]
        for o in outs:
            o.wait()
        return carry

    lax.fori_loop(0, _NCH, chunk, 0)


def kernel(tokens, emb_table, pos_table):
    tok = (tokens.astype(jnp.int32)
           .reshape(_B, _NW, _NCH, _K)
           .transpose(1, 0, 2, 3))
    mesh = plsc.VectorSubcoreMesh(core_axis_name="core",
                                  subcore_axis_name="subcore")
    f = pl.kernel(
        _emb_body,
        out_type=jax.ShapeDtypeStruct((_B, _S, _D), jnp.float32),
        mesh=mesh,
        scratch_types=[
            pltpu.VMEM((_B, _NCH, _K), jnp.int32),
            pltpu.VMEM((_K, _D), jnp.float32),
            pltpu.VMEM((_B, _K, _D), jnp.float32),
            pltpu.SemaphoreType.DMA,
            pltpu.SemaphoreType.DMA,
            pltpu.SemaphoreType.DMA,
            pltpu.SemaphoreType.DMA,
            pltpu.SemaphoreType.DMA,
        ],
    )
    return f(tok, emb_table, pos_table)


# speedup vs baseline: 1.6713x; 1.0777x over previous
import jax
import jax.numpy as jnp
from jax import lax
from jax.experimental import pallas as pl
from jax.experimental.pallas import tpu as pltpu
from jax.experimental.pallas import tpu_sc as plsc

_B, _S, _D = 4, 4096, 1024
_NW = 32
_SPW = _S // _NW
_K = 16
_NCH = _SPW // _K
_NBUF = 6


def _emb_body(tok_ref, emb_ref, pos_ref, out_ref, idx_v, pos_v, emb_v,
              *sems):
    gsems, osems = sems[:_NBUF], sems[_NBUF:]
    cid = lax.axis_index("core")
    sid = lax.axis_index("subcore")
    wid = sid * 2 + cid
    s_base = wid * _SPW

    pltpu.sync_copy(tok_ref.at[wid], idx_v)

    def gather(b, c, buf):
        return pltpu.async_copy(emb_ref.at[idx_v.at[b, c]], emb_v.at[buf],
                                gsems[buf])

    def out_copy(b, s0, buf):
        return pltpu.async_copy(emb_v.at[buf], out_ref.at[b, pl.ds(s0, _K)],
                                osems[buf])

    def add_pair(buf0, buf1):
        def row(r, carry):
            for h in range(2):
                base = h * (_D // 2)
                pv = [pos_v[r, pl.ds(base + j * 16, 16)]
                      for j in range(_D // 32)]
                for buf in (buf0, buf1):
                    for j in range(_D // 32):
                        sl = pl.ds(base + j * 16, 16)
                        emb_v[buf, r, sl] = emb_v[buf, r, sl] + pv[j]
            return carry
        lax.fori_loop(0, _K, row, 0)

    def chunk_pair(i, carry):
        cA = 2 * i
        cB = cA + 1
        sA = s_base + cA * _K
        sB = sA + _K
        gA01 = [gather(0, cA, 0), gather(1, cA, 1)]
        pltpu.sync_copy(pos_ref.at[pl.ds(sA, _K)], pos_v)
        gA01[0].wait()
        gA01[1].wait()
        gA23 = [gather(2, cA, 2), gather(3, cA, 3)]
        add_pair(0, 1)
        oA01 = [out_copy(0, sA, 0), out_copy(1, sA, 1)]
        gA23[0].wait()
        gA23[1].wait()
        gB01 = [gather(0, cB, 4), gather(1, cB, 5)]
        add_pair(2, 3)
        oA23 = [out_copy(2, sA, 2), out_copy(3, sA, 3)]
        gB01[0].wait()
        gB01[1].wait()
        pltpu.sync_copy(pos_ref.at[pl.ds(sB, _K)], pos_v)
        oA01[0].wait()
        oA01[1].wait()
        gB23 = [gather(2, cB, 0), gather(3, cB, 1)]
        add_pair(4, 5)
        oB01 = [out_copy(0, sB, 4), out_copy(1, sB, 5)]
        gB23[0].wait()
        gB23[1].wait()
        add_pair(0, 1)
        oB23 = [out_copy(2, sB, 0), out_copy(3, sB, 1)]
        for o in oA23 + oB01 + oB23:
            o.wait()
        return carry

    lax.fori_loop(0, _NCH // 2, chunk_pair, 0)


def kernel(tokens, emb_table, pos_table):
    tok = (tokens.astype(jnp.int32)
           .reshape(_B, _NW, _NCH, _K)
           .transpose(1, 0, 2, 3))
    mesh = plsc.VectorSubcoreMesh(core_axis_name="core",
                                  subcore_axis_name="subcore")
    f = pl.kernel(
        _emb_body,
        out_type=jax.ShapeDtypeStruct((_B, _S, _D), jnp.float32),
        mesh=mesh,
        scratch_types=[
            pltpu.VMEM((_B, _NCH, _K), jnp.int32),
            pltpu.VMEM((_K, _D), jnp.float32),
            pltpu.VMEM((_NBUF, _K, _D), jnp.float32),
        ] + [pltpu.SemaphoreType.DMA] * (2 * _NBUF),
    )
    return f(tok, emb_table, pos_table)
